# baseline (speedup 1.0000x reference)
import functools

import jax
import jax.numpy as jnp
from jax.experimental import pallas as pl
from jax.experimental.pallas import tpu as pltpu


def _body(ctx_ref, ctx_o_ref, w_ref, out_ref, m_ref, s_ref, lse_ref, co_ref,
          *, ngroup, nk, kt):
    g = pl.program_id(0)
    k = pl.program_id(1)
    nchunk = kt // 128

    wtile = w_ref[:, pl.ds(k * kt, kt)]

    @pl.when(g > 0)
    def _out():
        @pl.when(k == 0)
        def _make_operand():
            co_ref[...] = ctx_o_ref[...]
            nlse = -lse_ref[:, :1]
            hi = nlse.astype(jnp.bfloat16)
            lo = (nlse - hi.astype(jnp.float32)).astype(jnp.bfloat16)
            co_ref[:, 129:130] = hi
            co_ref[:, 130:131] = lo

        out_ref[...] = jax.lax.dot_general(
            co_ref[...], wtile,
            dimension_numbers=(((1,), (0,)), ((), ())),
            preferred_element_type=jnp.float32,
        )

    @pl.when(g < ngroup)
    def _stats():
        logits = jax.lax.dot_general(
            ctx_ref[...], wtile,
            dimension_numbers=(((1,), (0,)), ((), ())),
            preferred_element_type=jnp.float32,
        )

        @pl.when(k == 0)
        def _init():
            m_ref[...] = jnp.full_like(m_ref[...], -jnp.inf)
            s_ref[...] = jnp.zeros_like(s_ref[...])

        t = logits[:, 0:128]
        for c in range(1, nchunk):
            t = jnp.maximum(t, logits[:, c * 128:(c + 1) * 128])
        m_old = m_ref[...]
        m_new = jnp.maximum(m_old, t)
        acc = s_ref[...] * jnp.exp(m_old - m_new)
        for c in range(nchunk):
            acc = acc + jnp.exp(logits[:, c * 128:(c + 1) * 128] - m_new)
        s_ref[...] = acc
        m_ref[...] = m_new

        @pl.when(k == nk - 1)
        def _finalize():
            m = m_ref[...]
            s = s_ref[...]
            mrow = jnp.max(m, axis=1, keepdims=True)
            srow = jnp.sum(s * jnp.exp(m - mrow), axis=1, keepdims=True)
            lse = mrow + jnp.log(srow)
            lse_ref[...] = jnp.broadcast_to(lse, m.shape)


@jax.jit
def kernel(context, W, b):
    B, D = context.shape
    K = W.shape[1]
    KT = 5120
    NK = -(-K // KT)
    KP = NK * KT
    G = 8
    BT = B // G
    DA = 136

    ctx16 = jnp.concatenate(
        [
            context.astype(jnp.bfloat16),
            jnp.ones((B, 1), jnp.bfloat16),
            jnp.zeros((B, DA - D - 1), jnp.bfloat16),
        ],
        axis=1,
    )
    bpad = jnp.pad(b.reshape(1, K), ((0, 0), (0, KP - K)),
                   constant_values=-1e30).astype(jnp.bfloat16)
    W_aug = jnp.concatenate(
        [
            jnp.pad(W.astype(jnp.bfloat16), ((0, 0), (0, KP - K))),
            bpad,
            jnp.ones((2, KP), jnp.bfloat16),
            jnp.zeros((DA - D - 3, KP), jnp.bfloat16),
        ],
        axis=0,
    )

    return pl.pallas_call(
        functools.partial(_body, ngroup=G, nk=NK, kt=KT),
        grid=(G + 1, NK),
        in_specs=[
            pl.BlockSpec((BT, DA), lambda g, k: (jnp.minimum(g, G - 1), 0)),
            pl.BlockSpec((BT, DA), lambda g, k: (jnp.maximum(g - 1, 0), 0)),
            pl.BlockSpec((DA, KP), lambda g, k: (0, 0)),
        ],
        out_specs=pl.BlockSpec(
            (BT, KT),
            lambda g, k: (jnp.maximum(g - 1, 0), k * jnp.minimum(g, 1)),
        ),
        out_shape=jax.ShapeDtypeStruct((B, K), jnp.float32),
        scratch_shapes=[
            pltpu.VMEM((BT, 128), jnp.float32),
            pltpu.VMEM((BT, 128), jnp.float32),
            pltpu.VMEM((BT, 128), jnp.float32),
            pltpu.VMEM((BT, 136), jnp.bfloat16),
        ],
        compiler_params=pltpu.CompilerParams(
            dimension_semantics=("arbitrary", "arbitrary"),
        ),
    )(ctx16, ctx16, W_aug)


# speedup vs baseline: 1.0732x; 1.0732x over previous
import functools

import jax
import jax.numpy as jnp
from jax.experimental import pallas as pl
from jax.experimental.pallas import tpu as pltpu


def _body(ctx_ref, ctx_o_ref, w_ref, out_ref, m_ref, s_ref, lse_ref, co_ref,
          *, ngroup, nk, kt):
    g = pl.program_id(0)
    k = pl.program_id(1)
    nchunk = kt // 128

    wtile = w_ref[:, pl.ds(k * kt, kt)]

    @pl.when(g > 0)
    def _out():
        @pl.when(k == 0)
        def _make_operand():
            co_ref[...] = ctx_o_ref[...]
            nlse = -lse_ref[:, :1]
            hi = nlse.astype(jnp.bfloat16)
            lo = (nlse - hi.astype(jnp.float32)).astype(jnp.bfloat16)
            co_ref[:, 129:130] = hi
            co_ref[:, 130:131] = lo

        out_ref[...] = jax.lax.dot_general(
            co_ref[...], wtile,
            dimension_numbers=(((1,), (0,)), ((), ())),
            preferred_element_type=jnp.float32,
        )

    @pl.when(g < ngroup)
    def _stats():
        logits = jax.lax.dot_general(
            ctx_ref[...], wtile,
            dimension_numbers=(((1,), (0,)), ((), ())),
            preferred_element_type=jnp.float32,
        )

        @pl.when(k == 0)
        def _init():
            m_ref[...] = jnp.full_like(m_ref[...], -jnp.inf)
            s_ref[...] = jnp.zeros_like(s_ref[...])

        t = logits[:, 0:128]
        for c in range(1, nchunk):
            t = jnp.maximum(t, logits[:, c * 128:(c + 1) * 128])
        m_old = m_ref[...]
        m_new = jnp.maximum(m_old, t)
        acc = s_ref[...] * jnp.exp(m_old - m_new)
        for c in range(nchunk):
            acc = acc + jnp.exp(logits[:, c * 128:(c + 1) * 128] - m_new)
        s_ref[...] = acc
        m_ref[...] = m_new

        @pl.when(k == nk - 1)
        def _finalize():
            m = m_ref[...]
            s = s_ref[...]
            mrow = jnp.max(m, axis=1, keepdims=True)
            srow = jnp.sum(s * jnp.exp(m - mrow), axis=1, keepdims=True)
            lse = mrow + jnp.log(srow)
            lse_ref[...] = jnp.broadcast_to(lse, m.shape)


@jax.jit
def kernel(context, W, b):
    B, D = context.shape
    K = W.shape[1]
    KT = 5120
    NK = -(-K // KT)
    KP = NK * KT
    G = 4
    BT = B // G
    DA = 136

    ctx16 = jnp.concatenate(
        [
            context.astype(jnp.bfloat16),
            jnp.ones((B, 1), jnp.bfloat16),
            jnp.zeros((B, DA - D - 1), jnp.bfloat16),
        ],
        axis=1,
    )
    bpad = jnp.pad(b.reshape(1, K), ((0, 0), (0, KP - K)),
                   constant_values=-1e30).astype(jnp.bfloat16)
    W_aug = jnp.concatenate(
        [
            jnp.pad(W.astype(jnp.bfloat16), ((0, 0), (0, KP - K))),
            bpad,
            jnp.ones((2, KP), jnp.bfloat16),
            jnp.zeros((DA - D - 3, KP), jnp.bfloat16),
        ],
        axis=0,
    )

    return pl.pallas_call(
        functools.partial(_body, ngroup=G, nk=NK, kt=KT),
        grid=(G + 1, NK),
        in_specs=[
            pl.BlockSpec((BT, DA), lambda g, k: (jnp.minimum(g, G - 1), 0)),
            pl.BlockSpec((BT, DA), lambda g, k: (jnp.maximum(g - 1, 0), 0)),
            pl.BlockSpec((DA, KP), lambda g, k: (0, 0)),
        ],
        out_specs=pl.BlockSpec(
            (BT, KT),
            lambda g, k: (jnp.maximum(g - 1, 0), k * jnp.minimum(g, 1)),
        ),
        out_shape=jax.ShapeDtypeStruct((B, K), jnp.float32),
        scratch_shapes=[
            pltpu.VMEM((BT, 128), jnp.float32),
            pltpu.VMEM((BT, 128), jnp.float32),
            pltpu.VMEM((BT, 128), jnp.float32),
            pltpu.VMEM((BT, 136), jnp.bfloat16),
        ],
        compiler_params=pltpu.CompilerParams(
            dimension_semantics=("arbitrary", "arbitrary"),
        ),
    )(ctx16, ctx16, W_aug)
